# baseline (speedup 1.0000x reference)
import functools

import jax
import jax.numpy as jnp
from jax import lax
from jax.experimental import pallas as pl
from jax.experimental.pallas import tpu as pltpu
from jax.experimental.pallas import tpu_sc as plsc

NC = 2
NS = 16
NW = NC * NS
L = 16
CHUNK = 128


def _sc_aggregate(feat, src, dst, rc, with_cnt):
    n, d = feat.shape
    cpt = src.shape[0] // (NW * CHUNK)
    rpt = (-(-n // NS) + 7) // 8 * 8
    rpt_last = n - (NS - 1) * rpt
    nb = 2

    def body(feat_hbm, src_hbm, dst_hbm, out_hbm, *rest):
        if with_cnt:
            (cnt_hbm, is0, is1, id0, id1, rows0, rows1,
             sm0, sm1, ones_c, zb_v, acc_sh, cnt_sh) = rest
        else:
            (is0, is1, id0, id1, rows0, rows1, sm0, sm1, acc_sh) = rest
            cnt_hbm = ones_c = zb_v = cnt_sh = None
        rows_v = rows0
        bufs = [(rows0, is0, id0, sm0), (rows1, is1, id1, sm1)]
        cid = lax.axis_index("c")
        sid = lax.axis_index("s")
        wid = sid * NC + cid
        base = sid * rpt

        zrow = jnp.zeros((L,), jnp.float32)

        def zero_rows(i, _):
            for j in range(d // L):
                rows_v[i, pl.ds(j * L, L)] = zrow
            return 0

        lax.fori_loop(0, CHUNK, zero_rows, 0)
        if with_cnt:
            def fill_ones(i, _):
                ones_c[pl.ds(i * L, L)] = jnp.ones((L,), jnp.float32)
                return 0

            lax.fori_loop(0, CHUNK // L, fill_ones, 0)

            def zero_zb(i, _):
                zb_v[pl.ds(i * L, L)] = zrow
                return 0

            lax.fori_loop(0, zb_v.shape[0] // L, zero_zb, 0)

        def zero_slab(nr_total):
            for k in range(0, nr_total, CHUNK):
                nr = min(CHUNK, nr_total - k)
                pltpu.sync_copy(rows_v.at[pl.ds(0, nr)],
                                acc_sh.at[pl.ds(base + k, nr)])
            if with_cnt:
                pltpu.sync_copy(zb_v.at[pl.ds(0, nr_total)],
                                cnt_sh.at[pl.ds(base, nr_total)])

        if rpt == rpt_last:
            zero_slab(rpt)
        else:
            pl.when(sid < NS - 1)(lambda: zero_slab(rpt))
            pl.when(sid == NS - 1)(lambda: zero_slab(rpt_last))
        plsc.subcore_barrier()

        def gstart(j, rows, idx_s, idx_d, sem):
            off = (wid * cpt + j) * CHUNK
            pltpu.sync_copy(src_hbm.at[pl.ds(off, CHUNK)], idx_s)
            pltpu.sync_copy(dst_hbm.at[pl.ds(off, CHUNK)], idx_d)
            pltpu.make_async_copy(feat_hbm.at[idx_s], rows, sem).start()

        def gwait(rows, idx_s, sem):
            pltpu.make_async_copy(feat_hbm.at[idx_s], rows, sem).wait()

        def sstart(rows, idx_d, sem):
            pltpu.make_async_copy(rows, acc_sh.at[idx_d], sem).start(add=True)
            if with_cnt:
                pltpu.make_async_copy(
                    ones_c, cnt_sh.at[idx_d], sem).start(add=True)

        def sdrain(rows, idx_d, sem):
            pltpu.make_async_copy(rows, acc_sh.at[idx_d], sem).wait()
            if with_cnt:
                pltpu.make_async_copy(ones_c, cnt_sh.at[idx_d], sem).wait()

        for b in range(nb - 1):
            pl.when(wid * cpt + b < rc)(
                functools.partial(gstart, b, *bufs[b]))

        def visit(j, b):
            rows, idx_s, idx_d, sem = bufs[b]
            g = wid * cpt + j

            def _consume():
                gwait(rows, idx_s, sem)
                sstart(rows, idx_d, sem)

            pl.when(g < rc)(_consume)
            nbuf = bufs[(b + nb - 1) % nb]
            static = isinstance(j, int)
            if not static or j >= 1:
                pl.when(jnp.logical_and(j >= 1, g - 1 < rc))(
                    lambda: sdrain(nbuf[0], nbuf[2], nbuf[3]))
            if not static or j + nb - 1 < cpt:
                pl.when(jnp.logical_and(j + nb - 1 < cpt, g + nb - 1 < rc))(
                    lambda: gstart(j + nb - 1, *nbuf))

        def step(k, _):
            for b in range(nb):
                visit(k * nb + b, b)
            return 0

        lax.fori_loop(0, cpt // nb, step, 0)
        for j in range((cpt // nb) * nb, cpt):
            visit(j, j % nb)
        lbuf = bufs[(cpt - 1) % nb]
        pl.when(wid * cpt + cpt - 1 < rc)(
            functools.partial(sdrain, lbuf[0], lbuf[2], lbuf[3]))
        plsc.subcore_barrier()

        def write_slab(nr_total):
            pltpu.sync_copy(acc_sh.at[pl.ds(base, nr_total)],
                            out_hbm.at[cid, pl.ds(base, nr_total)])
            if with_cnt:
                pltpu.sync_copy(cnt_sh.at[pl.ds(base, nr_total)],
                                zb_v.at[pl.ds(0, nr_total)])
                pltpu.sync_copy(zb_v.at[pl.ds(0, nr_total)],
                                cnt_hbm.at[pl.ds(cid * n + base, nr_total)])

        if rpt == rpt_last:
            write_slab(rpt)
        else:
            pl.when(sid < NS - 1)(lambda: write_slab(rpt))
            pl.when(sid == NS - 1)(lambda: write_slab(rpt_last))

    out_type = [jax.ShapeDtypeStruct((NC, n, d), jnp.float32)]
    scratch = [pltpu.VMEM((CHUNK,), jnp.int32)] * 2
    scratch += [pltpu.VMEM((CHUNK,), jnp.int32)] * 2
    scratch += [pltpu.VMEM((CHUNK, d), jnp.float32)] * 2
    scratch += [pltpu.SemaphoreType.DMA] * 2
    if with_cnt:
        out_type.append(jax.ShapeDtypeStruct((NC * n,), jnp.float32))
        scratch.append(pltpu.VMEM((CHUNK,), jnp.float32))
        scratch.append(pltpu.VMEM(((rpt + L - 1) // L * L,), jnp.float32))
    scratch.append(pltpu.VMEM_SHARED((n, d), jnp.float32))
    if with_cnt:
        scratch.append(pltpu.VMEM_SHARED((n,), jnp.float32))

    mesh = plsc.VectorSubcoreMesh(core_axis_name="c", subcore_axis_name="s")
    k = pl.kernel(body, out_type=tuple(out_type), mesh=mesh,
                  scratch_types=tuple(scratch))
    return k(feat, src, dst)


def _tc_sage_body(p_ref, c_ref, x_ref, wl_ref, b_ref, wr_ref, o_ref, *, relu):
    s = p_ref[0] + p_ref[1]
    c = (c_ref[:, 0] + c_ref[:, 1])[:, None]
    mean = s / jnp.maximum(c, 1.0)
    r = (jnp.dot(mean, wl_ref[...], preferred_element_type=jnp.float32,
                 precision=lax.Precision.HIGHEST)
         + b_ref[...]
         + jnp.dot(x_ref[...], wr_ref[...], preferred_element_type=jnp.float32,
                   precision=lax.Precision.HIGHEST))
    o_ref[...] = jnp.maximum(r, 0.0) if relu else r


def _tc_sage(p, cnt, feat, wl_t, bl, wr_t, relu):
    n, d = feat.shape
    blk = next(b for b in range(min(1024, n), 0, -8)
               if b % 8 == 0 and n % b == 0)
    grid = (n // blk,)
    return pl.pallas_call(
        functools.partial(_tc_sage_body, relu=relu),
        grid=grid,
        in_specs=[
            pl.BlockSpec((NC, blk, d), lambda i: (0, i, 0)),
            pl.BlockSpec((blk, NC), lambda i: (i, 0)),
            pl.BlockSpec((blk, d), lambda i: (i, 0)),
            pl.BlockSpec((d, d), lambda i: (0, 0)),
            pl.BlockSpec((1, d), lambda i: (0, 0)),
            pl.BlockSpec((d, d), lambda i: (0, 0)),
        ],
        out_specs=pl.BlockSpec((blk, d), lambda i: (i, 0)),
        out_shape=jax.ShapeDtypeStruct((n, d), jnp.float32),
    )(p, cnt, feat, wl_t, bl, wr_t)


def kernel(x, edge_index, W1l, b1l, W1r, W2l, b2l, W2r):
    n, d = x.shape
    e = edge_index.shape[1]

    rc = -(-e // CHUNK)
    e_pad = -(-e // (NW * CHUNK)) * (NW * CHUNK)
    cpt = e_pad // (NW * CHUNK)
    kb = rc * CHUNK - e

    src = edge_index[0].astype(jnp.int32)
    dst = edge_index[1].astype(jnp.int32)
    if e_pad != e:
        src = jnp.pad(src, (0, e_pad - e))
        dst = jnp.pad(dst, (0, e_pad - e))

    w1l_t, w1r_t = W1l.T, W1r.T
    w2l_t, w2r_t = W2l.T, W2r.T
    b1 = b1l.reshape(1, d)
    b2 = b2l.reshape(1, d)
    c_star = ((rc - 1) // cpt) % NC

    p1, cnt = _sc_aggregate(x, src, dst, rc, with_cnt=True)
    if kb:
        p1 = p1.at[c_star, 0].add(-kb * x[0])
        cnt = cnt.at[c_star * n].add(-float(kb))
    cnt = cnt.reshape(NC, n).T
    h = _tc_sage(p1, cnt, x, w1l_t, b1, w1r_t, relu=True)
    (p2,) = _sc_aggregate(h, src, dst, rc, with_cnt=False)
    if kb:
        p2 = p2.at[c_star, 0].add(-kb * h[0])
    out = _tc_sage(p2, cnt, h, w2l_t, b2, w2r_t, relu=False)
    return out


# speedup vs baseline: 1.2446x; 1.2446x over previous
import functools

import jax
import jax.numpy as jnp
from jax import lax
from jax.experimental import pallas as pl
from jax.experimental.pallas import tpu as pltpu
from jax.experimental.pallas import tpu_sc as plsc

NC = 2
NS = 16
NW = NC * NS
L = 16
CHUNK = 128


def _sc_aggregate(feat, src, dst, rc, with_cnt):
    n, d = feat.shape
    cpt = src.shape[0] // (NW * CHUNK)
    rpt = (-(-n // NS) + 7) // 8 * 8
    rpt_last = n - (NS - 1) * rpt
    nb = 2

    def body(feat_hbm, src_hbm, dst_hbm, out_hbm, *rest):
        if with_cnt:
            (cnt_hbm, is0, is1, id0, id1, rows0, rows1,
             sm0, sm1, ones_c, zb_v, acc_sh, cnt_sh) = rest
        else:
            (is0, is1, id0, id1, rows0, rows1, sm0, sm1, acc_sh) = rest
            cnt_hbm = ones_c = zb_v = cnt_sh = None
        rows_v = rows0
        bufs = [(rows0, is0, id0, sm0), (rows1, is1, id1, sm1)]
        cid = lax.axis_index("c")
        sid = lax.axis_index("s")
        wid = sid * NC + cid
        base = sid * rpt

        zrow = jnp.zeros((L,), jnp.float32)

        def zero_rows(i, _):
            for j in range(d // L):
                rows_v[i, pl.ds(j * L, L)] = zrow
            return 0

        lax.fori_loop(0, CHUNK, zero_rows, 0)
        if with_cnt:
            def fill_ones(i, _):
                ones_c[pl.ds(i * L, L)] = jnp.ones((L,), jnp.float32)
                return 0

            lax.fori_loop(0, CHUNK // L, fill_ones, 0)

            def zero_zb(i, _):
                zb_v[pl.ds(i * L, L)] = zrow
                return 0

            lax.fori_loop(0, zb_v.shape[0] // L, zero_zb, 0)

        def zero_slab(nr_total):
            for k in range(0, nr_total, CHUNK):
                nr = min(CHUNK, nr_total - k)
                pltpu.sync_copy(rows_v.at[pl.ds(0, nr)],
                                acc_sh.at[pl.ds(base + k, nr)])
            if with_cnt:
                pltpu.sync_copy(zb_v.at[pl.ds(0, nr_total)],
                                cnt_sh.at[pl.ds(base, nr_total)])

        if rpt == rpt_last:
            zero_slab(rpt)
        else:
            pl.when(sid < NS - 1)(lambda: zero_slab(rpt))
            pl.when(sid == NS - 1)(lambda: zero_slab(rpt_last))
        plsc.subcore_barrier()

        def gstart(j, rows, idx_s, idx_d, sem):
            off = (wid * cpt + j) * CHUNK
            pltpu.sync_copy(src_hbm.at[pl.ds(off, CHUNK)], idx_s)
            pltpu.sync_copy(dst_hbm.at[pl.ds(off, CHUNK)], idx_d)
            pltpu.make_async_copy(feat_hbm.at[idx_s], rows, sem).start()

        pl.when(wid * cpt < rc)(functools.partial(gstart, 0, *bufs[0]))

        def visit(j, b):
            rows, idx_s, idx_d, sem = bufs[b]
            nbuf = bufs[(b + 1) % nb]
            g = wid * cpt + j
            if not isinstance(j, int) or j + 1 < cpt:
                pl.when(jnp.logical_and(j + 1 < cpt, g + 1 < rc))(
                    functools.partial(gstart, j + 1, *nbuf))

            def _consume():
                pltpu.make_async_copy(feat_hbm.at[idx_s], rows, sem).wait()
                pltpu.sync_copy(rows, acc_sh.at[idx_d], add=True)
                if with_cnt:
                    pltpu.sync_copy(ones_c, cnt_sh.at[idx_d], add=True)

            pl.when(g < rc)(_consume)

        def step(k, _):
            for b in range(nb):
                visit(k * nb + b, b)
            return 0

        lax.fori_loop(0, cpt // nb, step, 0)
        for j in range((cpt // nb) * nb, cpt):
            visit(j, j % nb)
        plsc.subcore_barrier()

        def write_slab(nr_total):
            pltpu.sync_copy(acc_sh.at[pl.ds(base, nr_total)],
                            out_hbm.at[cid, pl.ds(base, nr_total)])
            if with_cnt:
                pltpu.sync_copy(cnt_sh.at[pl.ds(base, nr_total)],
                                zb_v.at[pl.ds(0, nr_total)])
                pltpu.sync_copy(zb_v.at[pl.ds(0, nr_total)],
                                cnt_hbm.at[pl.ds(cid * n + base, nr_total)])

        if rpt == rpt_last:
            write_slab(rpt)
        else:
            pl.when(sid < NS - 1)(lambda: write_slab(rpt))
            pl.when(sid == NS - 1)(lambda: write_slab(rpt_last))

    out_type = [jax.ShapeDtypeStruct((NC, n, d), jnp.float32)]
    scratch = [pltpu.VMEM((CHUNK,), jnp.int32)] * 2
    scratch += [pltpu.VMEM((CHUNK,), jnp.int32)] * 2
    scratch += [pltpu.VMEM((CHUNK, d), jnp.float32)] * 2
    scratch += [pltpu.SemaphoreType.DMA] * 2
    if with_cnt:
        out_type.append(jax.ShapeDtypeStruct((NC * n,), jnp.float32))
        scratch.append(pltpu.VMEM((CHUNK,), jnp.float32))
        scratch.append(pltpu.VMEM(((rpt + L - 1) // L * L,), jnp.float32))
    scratch.append(pltpu.VMEM_SHARED((n, d), jnp.float32))
    if with_cnt:
        scratch.append(pltpu.VMEM_SHARED((n,), jnp.float32))

    mesh = plsc.VectorSubcoreMesh(core_axis_name="c", subcore_axis_name="s")
    k = pl.kernel(body, out_type=tuple(out_type), mesh=mesh,
                  scratch_types=tuple(scratch))
    return k(feat, src, dst)


def _tc_sage_body(p_ref, c_ref, x_ref, wl_ref, b_ref, wr_ref, o_ref, *, relu):
    s = p_ref[0] + p_ref[1]
    c = (c_ref[:, 0] + c_ref[:, 1])[:, None]
    mean = s / jnp.maximum(c, 1.0)
    r = (jnp.dot(mean, wl_ref[...], preferred_element_type=jnp.float32,
                 precision=lax.Precision.HIGHEST)
         + b_ref[...]
         + jnp.dot(x_ref[...], wr_ref[...], preferred_element_type=jnp.float32,
                   precision=lax.Precision.HIGHEST))
    o_ref[...] = jnp.maximum(r, 0.0) if relu else r


def _tc_sage(p, cnt, feat, wl_t, bl, wr_t, relu):
    n, d = feat.shape
    blk = next(b for b in range(min(1024, n), 0, -8)
               if b % 8 == 0 and n % b == 0)
    grid = (n // blk,)
    return pl.pallas_call(
        functools.partial(_tc_sage_body, relu=relu),
        grid=grid,
        in_specs=[
            pl.BlockSpec((NC, blk, d), lambda i: (0, i, 0)),
            pl.BlockSpec((blk, NC), lambda i: (i, 0)),
            pl.BlockSpec((blk, d), lambda i: (i, 0)),
            pl.BlockSpec((d, d), lambda i: (0, 0)),
            pl.BlockSpec((1, d), lambda i: (0, 0)),
            pl.BlockSpec((d, d), lambda i: (0, 0)),
        ],
        out_specs=pl.BlockSpec((blk, d), lambda i: (i, 0)),
        out_shape=jax.ShapeDtypeStruct((n, d), jnp.float32),
    )(p, cnt, feat, wl_t, bl, wr_t)


def kernel(x, edge_index, W1l, b1l, W1r, W2l, b2l, W2r):
    n, d = x.shape
    e = edge_index.shape[1]

    rc = -(-e // CHUNK)
    e_pad = -(-e // (NW * CHUNK)) * (NW * CHUNK)
    cpt = e_pad // (NW * CHUNK)
    kb = rc * CHUNK - e

    src = edge_index[0].astype(jnp.int32)
    dst = edge_index[1].astype(jnp.int32)
    if e_pad != e:
        src = jnp.pad(src, (0, e_pad - e))
        dst = jnp.pad(dst, (0, e_pad - e))

    w1l_t, w1r_t = W1l.T, W1r.T
    w2l_t, w2r_t = W2l.T, W2r.T
    b1 = b1l.reshape(1, d)
    b2 = b2l.reshape(1, d)
    c_star = ((rc - 1) // cpt) % NC

    p1, cnt = _sc_aggregate(x, src, dst, rc, with_cnt=True)
    if kb:
        p1 = p1.at[c_star, 0].add(-kb * x[0])
        cnt = cnt.at[c_star * n].add(-float(kb))
    cnt = cnt.reshape(NC, n).T
    h = _tc_sage(p1, cnt, x, w1l_t, b1, w1r_t, relu=True)
    (p2,) = _sc_aggregate(h, src, dst, rc, with_cnt=False)
    if kb:
        p2 = p2.at[c_star, 0].add(-kb * h[0])
    out = _tc_sage(p2, cnt, h, w2l_t, b2, w2r_t, relu=False)
    return out


# speedup vs baseline: 1.4579x; 1.1714x over previous
import functools

import jax
import jax.numpy as jnp
from jax import lax
from jax.experimental import pallas as pl
from jax.experimental.pallas import tpu as pltpu
from jax.experimental.pallas import tpu_sc as plsc

NC = 2
NS = 16
NW = NC * NS
L = 16
CHUNK = 128


def _sc_aggregate(feat, src, dst, rc, with_cnt):
    n, d = feat.shape
    cpt = -(-rc // NW)
    rpt = (-(-n // NS) + 7) // 8 * 8
    rpt_last = n - (NS - 1) * rpt
    nb = 2

    def body(feat_hbm, src_hbm, dst_hbm, out_hbm, *rest):
        if with_cnt:
            (cnt_hbm, is0, is1, id0, id1, rows0, rows1,
             sm0, sm1, im0, im1, ones_c, zb_v, acc_sh, cnt_sh) = rest
        else:
            (is0, is1, id0, id1, rows0, rows1,
             sm0, sm1, im0, im1, acc_sh) = rest
            cnt_hbm = ones_c = zb_v = cnt_sh = None
        rows_v = rows0
        bufs = [(rows0, is0, id0, sm0, im0), (rows1, is1, id1, sm1, im1)]
        cid = lax.axis_index("c")
        sid = lax.axis_index("s")
        wid = sid * NC + cid
        base = sid * rpt

        zrow = jnp.zeros((L,), jnp.float32)

        def zero_rows(i, _):
            for j in range(d // L):
                rows_v[i, pl.ds(j * L, L)] = zrow
            return 0

        lax.fori_loop(0, CHUNK, zero_rows, 0)
        if with_cnt:
            def fill_ones(i, _):
                ones_c[pl.ds(i * L, L)] = jnp.ones((L,), jnp.float32)
                return 0

            lax.fori_loop(0, CHUNK // L, fill_ones, 0)

            def zero_zb(i, _):
                zb_v[pl.ds(i * L, L)] = zrow
                return 0

            lax.fori_loop(0, zb_v.shape[0] // L, zero_zb, 0)

        def zero_slab(nr_total):
            for k in range(0, nr_total, CHUNK):
                nr = min(CHUNK, nr_total - k)
                pltpu.sync_copy(rows_v.at[pl.ds(0, nr)],
                                acc_sh.at[pl.ds(base + k, nr)])
            if with_cnt:
                pltpu.sync_copy(zb_v.at[pl.ds(0, nr_total)],
                                cnt_sh.at[pl.ds(base, nr_total)])

        if rpt == rpt_last:
            zero_slab(rpt)
        else:
            pl.when(sid < NS - 1)(lambda: zero_slab(rpt))
            pl.when(sid == NS - 1)(lambda: zero_slab(rpt_last))
        plsc.subcore_barrier()

        def goff(j):
            return (wid + NW * j) * CHUNK

        def idx_load(j, idx_s, idx_d, isem):
            pltpu.make_async_copy(
                src_hbm.at[pl.ds(goff(j), CHUNK)], idx_s, isem).start()
            pltpu.make_async_copy(
                dst_hbm.at[pl.ds(goff(j), CHUNK)], idx_d, isem).start()

        def idx_wait(j, idx_s, idx_d, isem):
            pltpu.make_async_copy(
                src_hbm.at[pl.ds(goff(j), CHUNK)], idx_s, isem).wait()
            pltpu.make_async_copy(
                dst_hbm.at[pl.ds(goff(j), CHUNK)], idx_d, isem).wait()

        def gstart(j, b):
            rows, idx_s, idx_d, sem, isem = bufs[b]
            idx_wait(j, idx_s, idx_d, isem)
            pltpu.make_async_copy(feat_hbm.at[idx_s], rows, sem).start()

        for b in range(nb):
            pl.when(wid + NW * b < rc)(
                functools.partial(idx_load, b, bufs[b][1], bufs[b][2],
                                  bufs[b][4]))
        pl.when(wid < rc)(functools.partial(gstart, 0, 0))

        def visit(j, b):
            rows, idx_s, idx_d, sem, isem = bufs[b]
            g = wid + NW * j
            if not isinstance(j, int) or j + 1 < cpt:
                pl.when(jnp.logical_and(j + 1 < cpt, g + NW < rc))(
                    functools.partial(gstart, j + 1, (b + 1) % nb))

            def _consume():
                pltpu.make_async_copy(feat_hbm.at[idx_s], rows, sem).wait()
                pltpu.sync_copy(rows, acc_sh.at[idx_d], add=True)
                if with_cnt:
                    pltpu.sync_copy(ones_c, cnt_sh.at[idx_d], add=True)

            pl.when(g < rc)(_consume)
            if not isinstance(j, int) or j + nb < cpt:
                pl.when(jnp.logical_and(j + nb < cpt, g + nb * NW < rc))(
                    functools.partial(idx_load, j + nb, idx_s, idx_d, isem))

        def step(k, _):
            for b in range(nb):
                visit(k * nb + b, b)
            return 0

        lax.fori_loop(0, cpt // nb, step, 0)
        for j in range((cpt // nb) * nb, cpt):
            visit(j, j % nb)
        plsc.subcore_barrier()

        def write_slab(nr_total):
            pltpu.sync_copy(acc_sh.at[pl.ds(base, nr_total)],
                            out_hbm.at[cid, pl.ds(base, nr_total)])
            if with_cnt:
                pltpu.sync_copy(cnt_sh.at[pl.ds(base, nr_total)],
                                zb_v.at[pl.ds(0, nr_total)])
                pltpu.sync_copy(zb_v.at[pl.ds(0, nr_total)],
                                cnt_hbm.at[pl.ds(cid * n + base, nr_total)])

        if rpt == rpt_last:
            write_slab(rpt)
        else:
            pl.when(sid < NS - 1)(lambda: write_slab(rpt))
            pl.when(sid == NS - 1)(lambda: write_slab(rpt_last))

    out_type = [jax.ShapeDtypeStruct((NC, n, d), jnp.float32)]
    scratch = [pltpu.VMEM((CHUNK,), jnp.int32)] * 2
    scratch += [pltpu.VMEM((CHUNK,), jnp.int32)] * 2
    scratch += [pltpu.VMEM((CHUNK, d), jnp.float32)] * 2
    scratch += [pltpu.SemaphoreType.DMA] * 2
    scratch += [pltpu.SemaphoreType.DMA] * 2
    if with_cnt:
        out_type.append(jax.ShapeDtypeStruct((NC * n,), jnp.float32))
        scratch.append(pltpu.VMEM((CHUNK,), jnp.float32))
        scratch.append(pltpu.VMEM(((rpt + L - 1) // L * L,), jnp.float32))
    scratch.append(pltpu.VMEM_SHARED((n, d), jnp.float32))
    if with_cnt:
        scratch.append(pltpu.VMEM_SHARED((n,), jnp.float32))

    mesh = plsc.VectorSubcoreMesh(core_axis_name="c", subcore_axis_name="s")
    k = pl.kernel(body, out_type=tuple(out_type), mesh=mesh,
                  scratch_types=tuple(scratch))
    return k(feat, src, dst)


def _tc_sage_body(p_ref, c_ref, x_ref, wl_ref, b_ref, wr_ref, o_ref, *, relu):
    s = p_ref[0] + p_ref[1]
    c = (c_ref[:, 0] + c_ref[:, 1])[:, None]
    mean = s / jnp.maximum(c, 1.0)
    r = (jnp.dot(mean, wl_ref[...], preferred_element_type=jnp.float32,
                 precision=lax.Precision.HIGHEST)
         + b_ref[...]
         + jnp.dot(x_ref[...], wr_ref[...], preferred_element_type=jnp.float32,
                   precision=lax.Precision.HIGHEST))
    o_ref[...] = jnp.maximum(r, 0.0) if relu else r


def _tc_sage(p, cnt, feat, wl_t, bl, wr_t, relu):
    n, d = feat.shape
    blk = next(b for b in range(min(1024, n), 0, -8)
               if b % 8 == 0 and n % b == 0)
    grid = (n // blk,)
    return pl.pallas_call(
        functools.partial(_tc_sage_body, relu=relu),
        grid=grid,
        in_specs=[
            pl.BlockSpec((NC, blk, d), lambda i: (0, i, 0)),
            pl.BlockSpec((blk, NC), lambda i: (i, 0)),
            pl.BlockSpec((blk, d), lambda i: (i, 0)),
            pl.BlockSpec((d, d), lambda i: (0, 0)),
            pl.BlockSpec((1, d), lambda i: (0, 0)),
            pl.BlockSpec((d, d), lambda i: (0, 0)),
        ],
        out_specs=pl.BlockSpec((blk, d), lambda i: (i, 0)),
        out_shape=jax.ShapeDtypeStruct((n, d), jnp.float32),
    )(p, cnt, feat, wl_t, bl, wr_t)


def kernel(x, edge_index, W1l, b1l, W1r, W2l, b2l, W2r):
    n, d = x.shape
    e = edge_index.shape[1]

    rc = -(-e // CHUNK)
    kb = rc * CHUNK - e

    src = edge_index[0].astype(jnp.int32)
    dst = edge_index[1].astype(jnp.int32)
    if kb:
        src = jnp.pad(src, (0, kb))
        dst = jnp.pad(dst, (0, kb))

    w1l_t, w1r_t = W1l.T, W1r.T
    w2l_t, w2r_t = W2l.T, W2r.T
    b1 = b1l.reshape(1, d)
    b2 = b2l.reshape(1, d)
    c_star = (rc - 1) % NC

    p1, cnt = _sc_aggregate(x, src, dst, rc, with_cnt=True)
    if kb:
        p1 = p1.at[c_star, 0].add(-kb * x[0])
        cnt = cnt.at[c_star * n].add(-float(kb))
    cnt = cnt.reshape(NC, n).T
    h = _tc_sage(p1, cnt, x, w1l_t, b1, w1r_t, relu=True)
    (p2,) = _sc_aggregate(h, src, dst, rc, with_cnt=False)
    if kb:
        p2 = p2.at[c_star, 0].add(-kb * h[0])
    out = _tc_sage(p2, cnt, h, w2l_t, b2, w2r_t, relu=False)
    return out


# speedup vs baseline: 1.5961x; 1.0948x over previous
import functools

import jax
import jax.numpy as jnp
from jax import lax
from jax.experimental import pallas as pl
from jax.experimental.pallas import tpu as pltpu
from jax.experimental.pallas import tpu_sc as plsc

NC = 2
NS = 16
NW = NC * NS
L = 16
CHUNK = 128


def _sc_aggregate(feat, ei, rc, with_cnt):
    n, d = feat.shape
    cpt = -(-rc // NW)
    rpt = (-(-n // NS) + 7) // 8 * 8
    rpt_last = n - (NS - 1) * rpt
    nb = 2

    def body(feat_hbm, ei_hbm, out_hbm, *rest):
        if with_cnt:
            (cnt_hbm, ix0, ix1, rows0, rows1,
             sm0, sm1, im0, im1, ones_c, zb_v, acc_sh, cnt_sh) = rest
        else:
            (ix0, ix1, rows0, rows1, sm0, sm1, im0, im1, acc_sh) = rest
            cnt_hbm = ones_c = zb_v = cnt_sh = None
        rows_v = rows0
        bufs = [(rows0, ix0, sm0, im0), (rows1, ix1, sm1, im1)]
        cid = lax.axis_index("c")
        sid = lax.axis_index("s")
        wid = sid * NC + cid
        base = sid * rpt

        zrow = jnp.zeros((L,), jnp.float32)

        def zero_rows(i, _):
            for j in range(d // L):
                rows_v[i, pl.ds(j * L, L)] = zrow
            return 0

        lax.fori_loop(0, CHUNK, zero_rows, 0)
        if with_cnt:
            def fill_ones(i, _):
                ones_c[pl.ds(i * L, L)] = jnp.ones((L,), jnp.float32)
                return 0

            lax.fori_loop(0, CHUNK // L, fill_ones, 0)

            def zero_zb(i, _):
                zb_v[pl.ds(i * L, L)] = zrow
                return 0

            lax.fori_loop(0, zb_v.shape[0] // L, zero_zb, 0)

        def zero_slab(nr_total):
            for k in range(0, nr_total, CHUNK):
                nr = min(CHUNK, nr_total - k)
                pltpu.sync_copy(rows_v.at[pl.ds(0, nr)],
                                acc_sh.at[pl.ds(base + k, nr)])
            if with_cnt:
                pltpu.sync_copy(zb_v.at[pl.ds(0, nr_total)],
                                cnt_sh.at[pl.ds(base, nr_total)])

        if rpt == rpt_last:
            zero_slab(rpt)
        else:
            pl.when(sid < NS - 1)(lambda: zero_slab(rpt))
            pl.when(sid == NS - 1)(lambda: zero_slab(rpt_last))
        plsc.subcore_barrier()

        def goff(j):
            return (wid + NW * j) * CHUNK

        def idx_load(j, ix, isem):
            pltpu.make_async_copy(
                ei_hbm.at[:, pl.ds(goff(j), CHUNK)], ix, isem).start()

        def idx_wait(j, ix, isem):
            pltpu.make_async_copy(
                ei_hbm.at[:, pl.ds(goff(j), CHUNK)], ix, isem).wait()

        def gstart(j, b):
            rows, ix, sem, isem = bufs[b]
            idx_wait(j, ix, isem)
            pltpu.make_async_copy(feat_hbm.at[ix.at[0]], rows, sem).start()

        for b in range(nb):
            pl.when(wid + NW * b < rc)(
                functools.partial(idx_load, b, bufs[b][1], bufs[b][3]))
        pl.when(wid < rc)(functools.partial(gstart, 0, 0))

        def visit(j, b):
            rows, ix, sem, isem = bufs[b]
            g = wid + NW * j
            if not isinstance(j, int) or j + 1 < cpt:
                pl.when(jnp.logical_and(j + 1 < cpt, g + NW < rc))(
                    functools.partial(gstart, j + 1, (b + 1) % nb))

            def _consume():
                pltpu.make_async_copy(feat_hbm.at[ix.at[0]], rows, sem).wait()
                pltpu.sync_copy(rows, acc_sh.at[ix.at[1]], add=True)
                if with_cnt:
                    pltpu.sync_copy(ones_c, cnt_sh.at[ix.at[1]], add=True)

            pl.when(g < rc)(_consume)
            if not isinstance(j, int) or j + nb < cpt:
                pl.when(jnp.logical_and(j + nb < cpt, g + nb * NW < rc))(
                    functools.partial(idx_load, j + nb, ix, isem))

        def step(k, _):
            for b in range(nb):
                visit(k * nb + b, b)
            return 0

        lax.fori_loop(0, cpt // nb, step, 0)
        for j in range((cpt // nb) * nb, cpt):
            visit(j, j % nb)
        plsc.subcore_barrier()

        def write_slab(nr_total):
            pltpu.sync_copy(acc_sh.at[pl.ds(base, nr_total)],
                            out_hbm.at[cid, pl.ds(base, nr_total)])
            if with_cnt:
                pltpu.sync_copy(cnt_sh.at[pl.ds(base, nr_total)],
                                zb_v.at[pl.ds(0, nr_total)])
                pltpu.sync_copy(zb_v.at[pl.ds(0, nr_total)],
                                cnt_hbm.at[pl.ds(cid * n + base, nr_total)])

        if rpt == rpt_last:
            write_slab(rpt)
        else:
            pl.when(sid < NS - 1)(lambda: write_slab(rpt))
            pl.when(sid == NS - 1)(lambda: write_slab(rpt_last))

    out_type = [jax.ShapeDtypeStruct((NC, n, d), jnp.float32)]
    scratch = [pltpu.VMEM((2, CHUNK), jnp.int32)] * 2
    scratch += [pltpu.VMEM((CHUNK, d), jnp.float32)] * 2
    scratch += [pltpu.SemaphoreType.DMA] * 2
    scratch += [pltpu.SemaphoreType.DMA] * 2
    if with_cnt:
        out_type.append(jax.ShapeDtypeStruct((NC * n,), jnp.float32))
        scratch.append(pltpu.VMEM((CHUNK,), jnp.float32))
        scratch.append(pltpu.VMEM(((rpt + L - 1) // L * L,), jnp.float32))
    scratch.append(pltpu.VMEM_SHARED((n, d), jnp.float32))
    if with_cnt:
        scratch.append(pltpu.VMEM_SHARED((n,), jnp.float32))

    mesh = plsc.VectorSubcoreMesh(core_axis_name="c", subcore_axis_name="s")
    k = pl.kernel(body, out_type=tuple(out_type), mesh=mesh,
                  scratch_types=tuple(scratch))
    return k(feat, ei)


def _matmul_t(a, w):
    return lax.dot_general(a, w, (((1,), (1,)), ((), ())),
                           preferred_element_type=jnp.float32)


def _tc_sage_body(p_ref, c_ref, x_ref, wl_ref, b_ref, wr_ref, o_ref, *, relu):
    s = p_ref[0] + p_ref[1]
    c = (c_ref[:, 0] + c_ref[:, 1])[:, None]
    mean = s / jnp.maximum(c, 1.0)
    r = _matmul_t(mean, wl_ref[...]) + b_ref[...] + _matmul_t(x_ref[...],
                                                              wr_ref[...])
    o_ref[...] = jnp.maximum(r, 0.0) if relu else r


def _tc_sage(p, cnt, feat, wl_t, bl, wr_t, relu):
    n, d = feat.shape
    blk = next(b for b in range(min(1024, n), 0, -8)
               if b % 8 == 0 and n % b == 0)
    grid = (n // blk,)
    return pl.pallas_call(
        functools.partial(_tc_sage_body, relu=relu),
        grid=grid,
        in_specs=[
            pl.BlockSpec((NC, blk, d), lambda i: (0, i, 0)),
            pl.BlockSpec((blk, NC), lambda i: (i, 0)),
            pl.BlockSpec((blk, d), lambda i: (i, 0)),
            pl.BlockSpec((d, d), lambda i: (0, 0)),
            pl.BlockSpec((1, d), lambda i: (0, 0)),
            pl.BlockSpec((d, d), lambda i: (0, 0)),
        ],
        out_specs=pl.BlockSpec((blk, d), lambda i: (i, 0)),
        out_shape=jax.ShapeDtypeStruct((n, d), jnp.float32),
    )(p, cnt, feat, wl_t, bl, wr_t)


def kernel(x, edge_index, W1l, b1l, W1r, W2l, b2l, W2r):
    n, d = x.shape
    e = edge_index.shape[1]

    rc = -(-e // CHUNK)
    kb = rc * CHUNK - e

    ei = edge_index.astype(jnp.int32)
    if kb:
        ei = jnp.pad(ei, ((0, 0), (0, kb)))

    b1 = b1l.reshape(1, d)
    b2 = b2l.reshape(1, d)
    c_star = (rc - 1) % NC

    p1, cnt = _sc_aggregate(x, ei, rc, with_cnt=True)
    if kb:
        p1 = p1.at[c_star, 0].add(-kb * x[0])
        cnt = cnt.at[c_star * n].add(-float(kb))
    cnt = cnt.reshape(NC, n).T
    h = _tc_sage(p1, cnt, x, W1l, b1, W1r, relu=True)
    (p2,) = _sc_aggregate(h, ei, rc, with_cnt=False)
    if kb:
        p2 = p2.at[c_star, 0].add(-kb * h[0])
    out = _tc_sage(p2, cnt, h, W2l, b2, W2r, relu=False)
    return out
